# Initial kernel scaffold; baseline (speedup 1.0000x reference)
import jax
import jax.numpy as jnp
from jax.experimental import pallas as pl


def kernel(x, W_enc, b_enc):
    raise NotImplementedError("write your pallas kernel here")


# speedup vs baseline: 6.1724x; 6.1724x over previous
import functools

import jax
import jax.numpy as jnp
from jax.experimental import pallas as pl
from jax.experimental.pallas import tpu as pltpu

TOPK = 32


def _body(x_ref, w_ref, b_ref, o_ref, *, bc: int):
    c = pl.program_id(1)
    nc = pl.num_programs(1)
    z = jnp.dot(x_ref[...], w_ref[...], preferred_element_type=jnp.float32)
    z = z + b_ref[...]
    o_ref[:, pl.ds(c * bc, bc)] = z

    @pl.when(c == nc - 1)
    def _finish():
        zf = o_ref[...]
        zc = zf
        thr = None
        for _ in range(TOPK):
            thr = jnp.max(zc, axis=1, keepdims=True)
            zc = jnp.where(zc >= thr, -jnp.inf, zc)
        o_ref[...] = jnp.where((zf >= thr) & (zf > 0), zf, 0.0)


@jax.jit
def kernel(x, W_enc, b_enc):
    M, K = x.shape
    _, N = W_enc.shape
    BR = min(128, M)
    BC = min(2048, N)
    b2 = b_enc.reshape(1, N)
    grid = (M // BR, N // BC)
    return pl.pallas_call(
        functools.partial(_body, bc=BC),
        grid=grid,
        in_specs=[
            pl.BlockSpec((BR, K), lambda r, c: (r, 0)),
            pl.BlockSpec((K, BC), lambda r, c: (0, c)),
            pl.BlockSpec((1, BC), lambda r, c: (0, c)),
        ],
        out_specs=pl.BlockSpec((BR, N), lambda r, c: (r, 0)),
        out_shape=jax.ShapeDtypeStruct((M, N), jnp.float32),
        compiler_params=pltpu.CompilerParams(
            dimension_semantics=("parallel", "arbitrary"),
        ),
    )(x, W_enc, b2)


# speedup vs baseline: 16.4199x; 2.6602x over previous
import functools

import jax
import jax.numpy as jnp
from jax.experimental import pallas as pl
from jax.experimental.pallas import tpu as pltpu

TOPK = 32
FOLD = 16


def _body(x_ref, w_ref, b_ref, o_ref, *, bc: int):
    c = pl.program_id(1)
    nc = pl.num_programs(1)
    z = jnp.dot(x_ref[...], w_ref[...], preferred_element_type=jnp.float32)
    z = z + b_ref[...]
    o_ref[:, pl.ds(c * bc, bc)] = z

    @pl.when(c == nc - 1)
    def _finish():
        zf = o_ref[...]
        br, n = zf.shape
        gw = n // FOLD
        neg = jnp.float32(-jnp.inf)
        m1 = jnp.full((br, gw), neg, dtype=jnp.float32)
        m2 = m1
        m3 = m1
        for k in range(FOLD):
            v = zf[:, k * gw:(k + 1) * gw]
            l1 = jnp.minimum(m1, v)
            m1 = jnp.maximum(m1, v)
            l2 = jnp.minimum(m2, l1)
            m2 = jnp.maximum(m2, l1)
            m3 = jnp.maximum(m3, l2)
        g = jnp.concatenate([m1, m2, m3], axis=1)
        thr = None
        for _ in range(TOPK):
            thr = jnp.max(g, axis=1, keepdims=True)
            g = jnp.where(g >= thr, neg, g)
        o_ref[...] = jnp.where((zf >= thr) & (zf > 0), zf, 0.0)


@jax.jit
def kernel(x, W_enc, b_enc):
    M, K = x.shape
    _, N = W_enc.shape
    BR = min(256, M)
    BC = min(2048, N)
    b2 = b_enc.reshape(1, N)
    grid = (M // BR, N // BC)
    return pl.pallas_call(
        functools.partial(_body, bc=BC),
        grid=grid,
        in_specs=[
            pl.BlockSpec((BR, K), lambda r, c: (r, 0)),
            pl.BlockSpec((K, BC), lambda r, c: (0, c)),
            pl.BlockSpec((1, BC), lambda r, c: (0, c)),
        ],
        out_specs=pl.BlockSpec((BR, N), lambda r, c: (r, 0)),
        out_shape=jax.ShapeDtypeStruct((M, N), jnp.float32),
        compiler_params=pltpu.CompilerParams(
            dimension_semantics=("parallel", "arbitrary"),
        ),
    )(x, W_enc, b2)


# speedup vs baseline: 16.4759x; 1.0034x over previous
import functools

import jax
import jax.numpy as jnp
from jax.experimental import pallas as pl
from jax.experimental.pallas import tpu as pltpu

TOPK = 32


def _body(x_ref, w_ref, b_ref, o_ref, m1_ref, m2_ref, m3_ref, *, bc: int,
          gw: int):
    c = pl.program_id(1)
    nc = pl.num_programs(1)
    z = jnp.dot(x_ref[...], w_ref[...], preferred_element_type=jnp.float32)
    z = z + b_ref[...]
    o_ref[:, pl.ds(c * bc, bc)] = z

    @pl.when(c == 0)
    def _init():
        neg = jnp.full(m1_ref.shape, -jnp.inf, dtype=jnp.float32)
        m1_ref[...] = neg
        m2_ref[...] = neg
        m3_ref[...] = neg

    m1 = m1_ref[...]
    m2 = m2_ref[...]
    m3 = m3_ref[...]
    for k in range(bc // gw):
        v = z[:, k * gw:(k + 1) * gw]
        l1 = jnp.minimum(m1, v)
        m1 = jnp.maximum(m1, v)
        l2 = jnp.minimum(m2, l1)
        m2 = jnp.maximum(m2, l1)
        m3 = jnp.maximum(m3, l2)
    m1_ref[...] = m1
    m2_ref[...] = m2
    m3_ref[...] = m3

    @pl.when(c == nc - 1)
    def _finish():
        neg = jnp.float32(-jnp.inf)
        g = jnp.concatenate([m1, m2, m3], axis=1)
        hw = g.shape[1] // 24
        h1 = jnp.full((g.shape[0], hw), neg, dtype=jnp.float32)
        h2 = h1
        h3 = h1
        h4 = h1
        h5 = h1
        for k in range(24):
            v = g[:, k * hw:(k + 1) * hw]
            l1 = jnp.minimum(h1, v)
            h1 = jnp.maximum(h1, v)
            l2 = jnp.minimum(h2, l1)
            h2 = jnp.maximum(h2, l1)
            l3 = jnp.minimum(h3, l2)
            h3 = jnp.maximum(h3, l2)
            l4 = jnp.minimum(h4, l3)
            h4 = jnp.maximum(h4, l3)
            h5 = jnp.maximum(h5, l4)
        hh = jnp.concatenate([h1, h2, h3, h4, h5], axis=1)
        thr = None
        for i in range(TOPK):
            thr = jnp.max(hh, axis=1, keepdims=True)
            if i < TOPK - 1:
                hh = jnp.where(hh >= thr, neg, hh)
        zf = o_ref[...]
        o_ref[...] = jnp.where((zf >= thr) & (zf > 0), zf, 0.0)


@jax.jit
def kernel(x, W_enc, b_enc):
    M, K = x.shape
    _, N = W_enc.shape
    BR = min(256, M)
    BC = min(1024, N)
    GW = N // 16
    b2 = b_enc.reshape(1, N)
    grid = (M // BR, N // BC)
    return pl.pallas_call(
        functools.partial(_body, bc=BC, gw=GW),
        grid=grid,
        in_specs=[
            pl.BlockSpec((BR, K), lambda r, c: (r, 0)),
            pl.BlockSpec((K, BC), lambda r, c: (0, c)),
            pl.BlockSpec((1, BC), lambda r, c: (0, c)),
        ],
        out_specs=pl.BlockSpec((BR, N), lambda r, c: (r, 0)),
        out_shape=jax.ShapeDtypeStruct((M, N), jnp.float32),
        scratch_shapes=[
            pltpu.VMEM((BR, GW), jnp.float32),
            pltpu.VMEM((BR, GW), jnp.float32),
            pltpu.VMEM((BR, GW), jnp.float32),
        ],
        compiler_params=pltpu.CompilerParams(
            dimension_semantics=("parallel", "arbitrary"),
        ),
    )(x, W_enc, b2)
